# baseline (speedup 1.0000x reference)
import jax
import jax.numpy as jnp
from jax.experimental import pallas as pl
from jax.experimental.pallas import tpu as pltpu

_N_CODES = 20
_REPEAT = 10
_NBUF_IN = 3
_NBUF_OUT = 2
_MAXC = 10
_SIZES = [2, 3, 5] + [10] * 18 + [5, 3, 2]
_OFFS = [sum(_SIZES[:k]) for k in range(len(_SIZES))]
assert sum(_SIZES) == _N_CODES * _REPEAT


def _wcol(wt, code):
    mask = (jax.lax.broadcasted_iota(jnp.int32, wt.shape, 1) == code)
    return jnp.sum(jnp.where(mask, wt, 0.0), axis=1, keepdims=True)


def _body(x_hbm, wt_ref, o_hbm, xbuf, obuf, in_sems, out_sems):
    wt = wt_ref[...]
    n = len(_SIZES)

    def in_copy(k):
        slot = k % _NBUF_IN
        return pltpu.make_async_copy(
            x_hbm.at[pl.ds(_OFFS[k], _SIZES[k])],
            xbuf.at[slot, pl.ds(0, _SIZES[k])],
            in_sems.at[slot])

    def out_copy(k):
        slot = k % _NBUF_OUT
        return pltpu.make_async_copy(
            obuf.at[slot, pl.ds(0, _SIZES[k])],
            o_hbm.at[pl.ds(_OFFS[k], _SIZES[k])],
            out_sems.at[slot])

    for k in range(_NBUF_IN):
        in_copy(k).start()

    for k in range(n):
        si = k % _NBUF_IN
        so = k % _NBUF_OUT
        in_copy(k).wait()
        if k >= _NBUF_OUT:
            out_copy(k - _NBUF_OUT).wait()
        code = _OFFS[k] // _REPEAT
        obuf[so, : _SIZES[k]] = xbuf[si, : _SIZES[k]] + _wcol(wt, code)[None]
        out_copy(k).start()
        if k + _NBUF_IN < n:
            in_copy(k + _NBUF_IN).start()

    for k in range(n - _NBUF_OUT, n):
        out_copy(k).wait()


def kernel(X, W):
    B, T, D = X.shape
    Xt = jnp.transpose(X, (1, 2, 0))
    Wt = jnp.transpose(W)
    out_t = pl.pallas_call(
        _body,
        in_specs=[
            pl.BlockSpec(memory_space=pltpu.MemorySpace.HBM),
            pl.BlockSpec(memory_space=pltpu.MemorySpace.VMEM),
        ],
        out_specs=pl.BlockSpec(memory_space=pltpu.MemorySpace.HBM),
        out_shape=jax.ShapeDtypeStruct((T, D, B), X.dtype),
        scratch_shapes=[
            pltpu.VMEM((_NBUF_IN, _MAXC, D, B), jnp.float32),
            pltpu.VMEM((_NBUF_OUT, _MAXC, D, B), jnp.float32),
            pltpu.SemaphoreType.DMA((_NBUF_IN,)),
            pltpu.SemaphoreType.DMA((_NBUF_OUT,)),
        ],
    )(Xt, Wt)
    return jnp.transpose(out_t, (2, 0, 1))


# speedup vs baseline: 1.0022x; 1.0022x over previous
import jax
import jax.numpy as jnp
from jax.experimental import pallas as pl

_N_CODES = 20
_REPEAT = 10


def _body(x_ref, wt_ref, o_ref):
    i = pl.program_id(0)
    wt = wt_ref[...]
    mask = (jax.lax.broadcasted_iota(jnp.int32, wt.shape, 1) == i)
    wcol = jnp.sum(jnp.where(mask, wt, 0.0), axis=1, keepdims=True)
    o_ref[...] = x_ref[...] + wcol[None, :, :]


def kernel(X, W):
    B, T, D = X.shape
    Xt = jnp.transpose(X, (1, 2, 0))
    Wt = jnp.transpose(W)
    out_t = pl.pallas_call(
        _body,
        grid=(_N_CODES,),
        in_specs=[
            pl.BlockSpec((_REPEAT, D, B), lambda i: (i, 0, 0)),
            pl.BlockSpec((D, _N_CODES), lambda i: (0, 0)),
        ],
        out_specs=pl.BlockSpec((_REPEAT, D, B), lambda i: (i, 0, 0)),
        out_shape=jax.ShapeDtypeStruct((T, D, B), X.dtype),
    )(Xt, Wt)
    return jnp.transpose(out_t, (2, 0, 1))
